# Initial kernel scaffold; baseline (speedup 1.0000x reference)
import jax
import jax.numpy as jnp
from jax.experimental import pallas as pl


def kernel(input_ids, embedding_weight):
    raise NotImplementedError("write your pallas kernel here")


# speedup vs baseline: 8.2444x; 8.2444x over previous
import functools

import jax
import jax.numpy as jnp
from jax import lax
from jax.experimental import pallas as pl
from jax.experimental.pallas import tpu as pltpu
from jax.experimental.pallas import tpu_sc as plsc

_NC = 2
_NS = 16
_NW = _NC * _NS


@functools.lru_cache(maxsize=None)
def _build_gather(V, D, N, C):
    assert N % _NW == 0
    b_per_w = N // _NW
    assert b_per_w % C == 0 and C % 8 == 0
    nchunks = b_per_w // C
    assert nchunks % 2 == 0

    mesh = plsc.VectorSubcoreMesh(
        core_axis_name="c", subcore_axis_name="s",
        num_cores=_NC, num_subcores=_NS,
    )

    @functools.partial(
        pl.kernel,
        mesh=mesh,
        out_type=jax.ShapeDtypeStruct((N, D), jnp.float32),
        scratch_types=[
            pltpu.VMEM((2, C), jnp.int32),
            pltpu.VMEM((2, C, D), jnp.float32),
            pltpu.SemaphoreType.DMA,
            pltpu.SemaphoreType.DMA,
        ],
    )
    def k(table_hbm, idx_hbm, out_hbm, idx_v, rows_v, sem0, sem1):
        wid = lax.axis_index("s") * _NC + lax.axis_index("c")
        base = wid * b_per_w
        sems = (sem0, sem1)

        def issue(g, b):
            off = base + g * C
            pltpu.sync_copy(idx_hbm.at[pl.ds(off, C)], idx_v.at[b])
            pltpu.make_async_copy(
                table_hbm.at[idx_v.at[b]], rows_v.at[b], sems[b]
            ).start()

        def drain_store(g, b):
            off = base + g * C
            pltpu.make_async_copy(
                table_hbm.at[idx_v.at[b]], rows_v.at[b], sems[b]
            ).wait()
            pltpu.sync_copy(rows_v.at[b], out_hbm.at[pl.ds(off, C)])

        issue(0, 0)

        def pair_body(i2, carry):
            for b in (0, 1):
                g = i2 * 2 + b

                @pl.when(g + 1 < nchunks)
                def _():
                    issue(g + 1, 1 - b)

                drain_store(g, b)
            return carry

        lax.fori_loop(0, nchunks // 2, pair_body, 0)

    return k


def kernel(input_ids, embedding_weight):
    B, S = input_ids.shape
    V, D = embedding_weight.shape
    N = B * S
    idx = input_ids.reshape(N).astype(jnp.int32)
    out = _build_gather(V, D, N, 128)(embedding_weight, idx)
    return out.reshape(B, S, D)


# speedup vs baseline: 9.1463x; 1.1094x over previous
import functools

import jax
import jax.numpy as jnp
from jax import lax
from jax.experimental import pallas as pl
from jax.experimental.pallas import tpu as pltpu
from jax.experimental.pallas import tpu_sc as plsc

_NC = 2
_NS = 16
_NW = _NC * _NS


@functools.lru_cache(maxsize=None)
def _build_gather(V, D, N, C):
    assert N % _NW == 0
    b_per_w = N // _NW
    assert b_per_w % C == 0 and C % 8 == 0
    nchunks = b_per_w // C
    assert nchunks % 2 == 0

    NBUF = 4
    assert nchunks % NBUF == 0 and nchunks >= NBUF

    mesh = plsc.VectorSubcoreMesh(
        core_axis_name="c", subcore_axis_name="s",
        num_cores=_NC, num_subcores=_NS,
    )

    @functools.partial(
        pl.kernel,
        mesh=mesh,
        out_type=jax.ShapeDtypeStruct((N, D), jnp.float32),
        scratch_types=[
            pltpu.VMEM((NBUF, C), jnp.int32),
            pltpu.VMEM((NBUF, C, D), jnp.float32),
            [pltpu.SemaphoreType.DMA] * NBUF,
            [pltpu.SemaphoreType.DMA] * NBUF,
        ],
    )
    def k(table_hbm, idx_hbm, out_hbm, idx_v, rows_v, gsem, ssem):
        wid = lax.axis_index("s") * _NC + lax.axis_index("c")
        base = wid * b_per_w

        def issue(j, b):
            off = base + j * C
            pltpu.sync_copy(idx_hbm.at[pl.ds(off, C)], idx_v.at[b])
            pltpu.make_async_copy(
                table_hbm.at[idx_v.at[b]], rows_v.at[b], gsem[b]
            ).start()

        def wait_gather(b):
            pltpu.make_async_copy(
                table_hbm.at[idx_v.at[b]], rows_v.at[b], gsem[b]
            ).wait()

        def store(j, b):
            off = base + j * C
            return pltpu.make_async_copy(
                rows_v.at[b], out_hbm.at[pl.ds(off, C)], ssem[b]
            )

        for b in range(NBUF - 1):
            issue(b, b)

        def ring_body(i, carry):
            for bb in range(NBUF):
                j = i * NBUF + bb
                jn = j + NBUF - 1
                bn = (bb + NBUF - 1) % NBUF

                @pl.when(jn < nchunks)
                def _():
                    @pl.when(jn >= NBUF)
                    def _():
                        store(jn - NBUF, bn).wait()

                    issue(jn, bn)

                wait_gather(bb)
                store(j, bb).start()
            return carry

        lax.fori_loop(0, nchunks // NBUF, ring_body, 0)

        for j in range(nchunks - NBUF, nchunks):
            store(j, j % NBUF).wait()

    return k


def kernel(input_ids, embedding_weight):
    B, S = input_ids.shape
    V, D = embedding_weight.shape
    N = B * S
    idx = input_ids.reshape(N).astype(jnp.int32)
    out = _build_gather(V, D, N, 128)(embedding_weight, idx)
    return out.reshape(B, S, D)


# speedup vs baseline: 9.2504x; 1.0114x over previous
import functools

import jax
import jax.numpy as jnp
from jax import lax
from jax.experimental import pallas as pl
from jax.experimental.pallas import tpu as pltpu
from jax.experimental.pallas import tpu_sc as plsc

_NC = 2
_NS = 16
_NW = _NC * _NS


@functools.lru_cache(maxsize=None)
def _build_gather(V, D, N, C):
    assert N % _NW == 0
    b_per_w = N // _NW
    assert b_per_w % C == 0 and C % 8 == 0
    nchunks = b_per_w // C
    assert nchunks % 2 == 0

    NBUF = 5
    assert nchunks % NBUF == 0 and nchunks >= NBUF

    mesh = plsc.VectorSubcoreMesh(
        core_axis_name="c", subcore_axis_name="s",
        num_cores=_NC, num_subcores=_NS,
    )

    @functools.partial(
        pl.kernel,
        mesh=mesh,
        out_type=jax.ShapeDtypeStruct((N, D), jnp.float32),
        scratch_types=[
            pltpu.VMEM((b_per_w,), jnp.int32),
            pltpu.VMEM((NBUF, C, D), jnp.float32),
            [pltpu.SemaphoreType.DMA] * NBUF,
            [pltpu.SemaphoreType.DMA] * NBUF,
        ],
    )
    def k(table_hbm, idx_hbm, out_hbm, idx_v, rows_v, gsem, ssem):
        wid = lax.axis_index("s") * _NC + lax.axis_index("c")
        base = wid * b_per_w
        pltpu.sync_copy(idx_hbm.at[pl.ds(base, b_per_w)], idx_v)

        def gather(j, b):
            return pltpu.make_async_copy(
                table_hbm.at[idx_v.at[pl.ds(j * C, C)]], rows_v.at[b], gsem[b]
            )

        def issue(j, b):
            gather(j, b).start()

        def wait_gather(j, b):
            gather(j, b).wait()

        def store(j, b):
            off = base + j * C
            return pltpu.make_async_copy(
                rows_v.at[b], out_hbm.at[pl.ds(off, C)], ssem[b]
            )

        for b in range(NBUF - 1):
            issue(b, b)

        def ring_body(i, carry):
            for bb in range(NBUF):
                j = i * NBUF + bb
                jn = j + NBUF - 1
                bn = (bb + NBUF - 1) % NBUF

                @pl.when(jn < nchunks)
                def _():
                    @pl.when(jn >= NBUF)
                    def _():
                        store(jn - NBUF, bn).wait()

                    issue(jn, bn)

                wait_gather(j, bb)
                store(j, bb).start()
            return carry

        lax.fori_loop(0, nchunks // NBUF, ring_body, 0)

        for j in range(nchunks - NBUF, nchunks):
            store(j, j % NBUF).wait()

    return k


def kernel(input_ids, embedding_weight):
    B, S = input_ids.shape
    V, D = embedding_weight.shape
    N = B * S
    idx = input_ids.reshape(N).astype(jnp.int32)
    out = _build_gather(V, D, N, 128)(embedding_weight, idx)
    return out.reshape(B, S, D)


# speedup vs baseline: 9.2526x; 1.0002x over previous
import functools

import jax
import jax.numpy as jnp
from jax import lax
from jax.experimental import pallas as pl
from jax.experimental.pallas import tpu as pltpu
from jax.experimental.pallas import tpu_sc as plsc

_NC = 2
_NS = 16
_NW = _NC * _NS


@functools.lru_cache(maxsize=None)
def _build_gather(V, D, N, C):
    assert N % _NW == 0
    b_per_w = N // _NW
    assert b_per_w % C == 0 and C % 8 == 0
    nchunks = b_per_w // C
    assert nchunks % 2 == 0

    NBUF = 5
    assert nchunks % NBUF == 0 and nchunks >= NBUF

    mesh = plsc.VectorSubcoreMesh(
        core_axis_name="c", subcore_axis_name="s",
        num_cores=_NC, num_subcores=_NS,
    )

    @functools.partial(
        pl.kernel,
        mesh=mesh,
        out_type=jax.ShapeDtypeStruct((N, D), jnp.float32),
        scratch_types=[
            pltpu.VMEM((b_per_w,), jnp.int32),
            pltpu.VMEM((NBUF, C, D), jnp.float32),
            pltpu.VMEM((NBUF, C), jnp.int32),
            [pltpu.SemaphoreType.DMA] * NBUF,
            [pltpu.SemaphoreType.DMA] * NBUF,
        ],
    )
    def k(table_hbm, idx_hbm, out_hbm, idx_v, rows_v, pos_v, gsem, ssem):
        wid = lax.axis_index("s") * _NC + lax.axis_index("c")
        base = wid * b_per_w
        pltpu.sync_copy(idx_hbm.at[pl.ds(base, b_per_w)], idx_v)

        def gather(j, b):
            return pltpu.make_async_copy(
                table_hbm.at[idx_v.at[pl.ds(j * C, C)]], rows_v.at[b], gsem[b]
            )

        def issue(j, b):
            gather(j, b).start()

        def wait_gather(j, b):
            gather(j, b).wait()

        lane = jnp.arange(16, dtype=jnp.int32)

        def fill_pos(j, b):
            off = base + j * C
            for kk in range(C // 16):
                pos_v[b, pl.ds(kk * 16, 16)] = off + kk * 16 + lane

        def store(j, b):
            del j
            return pltpu.make_async_copy(
                rows_v.at[b], out_hbm.at[pos_v.at[b]], ssem[b]
            )

        for b in range(NBUF - 1):
            issue(b, b)

        def ring_body(i, carry):
            for bb in range(NBUF):
                j = i * NBUF + bb
                jn = j + NBUF - 1
                bn = (bb + NBUF - 1) % NBUF

                @pl.when(jn < nchunks)
                def _():
                    @pl.when(jn >= NBUF)
                    def _():
                        store(jn - NBUF, bn).wait()

                    issue(jn, bn)

                wait_gather(j, bb)
                fill_pos(j, bb)
                store(j, bb).start()
            return carry

        lax.fori_loop(0, nchunks // NBUF, ring_body, 0)

        for j in range(nchunks - NBUF, nchunks):
            store(j, j % NBUF).wait()

    return k


def kernel(input_ids, embedding_weight):
    B, S = input_ids.shape
    V, D = embedding_weight.shape
    N = B * S
    idx = input_ids.reshape(N).astype(jnp.int32)
    out = _build_gather(V, D, N, 128)(embedding_weight, idx)
    return out.reshape(B, S, D)
